# baseline (speedup 1.0000x reference)
import functools

import jax
import jax.numpy as jnp
from jax import lax
from jax.experimental import pallas as pl
from jax.experimental.pallas import tpu as pltpu
from jax.experimental.pallas import tpu_sc as plsc

_N = 16384
_C = 1000
_GAMMA = 2.0
_B = 1024
_NW = 32
_BPW = _N // _NW


@functools.lru_cache(maxsize=None)
def _sc_alpha_gather_fn():
    mesh = plsc.VectorSubcoreMesh(core_axis_name="c", subcore_axis_name="s")

    @functools.partial(
        pl.kernel,
        out_type=jax.ShapeDtypeStruct((_N,), jnp.float32),
        mesh=mesh,
        scratch_types=[
            pltpu.VMEM((_BPW,), jnp.int32),
            pltpu.VMEM((_BPW,), jnp.float32),
            pltpu.SemaphoreType.DMA,
        ],
    )
    def sc_gather(t_hbm, a_hbm, out_hbm, idx_v, rows_v, sem):
        wid = lax.axis_index("s") * 2 + lax.axis_index("c")
        base = wid * _BPW
        pltpu.sync_copy(t_hbm.at[pl.ds(base, _BPW)], idx_v)
        pltpu.async_copy(a_hbm.at[idx_v], rows_v, sem).wait()
        pltpu.sync_copy(rows_v, out_hbm.at[pl.ds(base, _BPW)])

    return sc_gather


def _focal_body(x_ref, t_ref, a_ref, out_ref):
    i = pl.program_id(0)
    nb = pl.num_programs(0)
    x = x_ref[...]
    t = t_ref[0, 0, :]
    at = a_ref[0, 0, :]

    m = jnp.max(x, axis=0)
    e = jnp.exp(x - m[None, :])

    iota = jax.lax.broadcasted_iota(jnp.int32, x.shape, 0)
    onehot = iota == t[None, :]
    em = jnp.where(onehot, e, 0.0)

    ones = jnp.ones((1, _C), jnp.float32)
    s = jax.lax.dot_general(ones, e, (((1,), (0,)), ((), ())),
                            preferred_element_type=jnp.float32)
    pe = jax.lax.dot_general(ones, em, (((1,), (0,)), ((), ())),
                             preferred_element_type=jnp.float32)

    p = pe / s
    logp = jnp.log(p)
    omp = 1.0 - p
    loss = -at[None, :] * (omp * omp) * logp
    bsum = jnp.sum(loss, keepdims=True).reshape(1, 1)

    @pl.when(i == 0)
    def _init():
        out_ref[...] = jnp.zeros((1, 1), jnp.float32)

    acc = out_ref[...] + bsum
    out_ref[...] = jnp.where(i == nb - 1, acc * (1.0 / _N), acc)


@jax.jit
def kernel(inputs, targets, alpha):
    nb = _N // _B
    at = _sc_alpha_gather_fn()(targets, alpha.reshape(_C))
    xt_view = inputs.T
    t3 = targets.reshape(nb, 1, _B)
    at3 = at.reshape(nb, 1, _B)
    out = pl.pallas_call(
        _focal_body,
        grid=(nb,),
        in_specs=[
            pl.BlockSpec((_C, _B), lambda i: (0, i)),
            pl.BlockSpec((1, 1, _B), lambda i: (i, 0, 0)),
            pl.BlockSpec((1, 1, _B), lambda i: (i, 0, 0)),
        ],
        out_specs=pl.BlockSpec((1, 1), lambda i: (0, 0)),
        out_shape=jax.ShapeDtypeStruct((1, 1), jnp.float32),
    )(xt_view, t3, at3)
    return out[0, 0]


# speedup vs baseline: 1.8149x; 1.8149x over previous
import functools

import jax
import jax.numpy as jnp
from jax.experimental import pallas as pl
from jax.experimental.pallas import tpu as pltpu

_N = 16384
_C = 1000
_GAMMA = 2.0
_B = 1024


def _focal_body(x_ref, t_ref, a_ref, out_ref):
    i = pl.program_id(0)
    nb = pl.num_programs(0)
    x = x_ref[...]
    t = t_ref[0, 0, :]
    a2 = a_ref[...]

    m = jnp.max(x, axis=0)
    e = jnp.exp(x - m[None, :])

    iota = jax.lax.broadcasted_iota(jnp.int32, x.shape, 0)
    onehot = iota == t[None, :]
    em = jnp.where(onehot, e, 0.0)

    ones = jnp.ones((1, _C), jnp.float32)
    s = jax.lax.dot_general(ones, e, (((1,), (0,)), ((), ())),
                            preferred_element_type=jnp.float32)
    pe = jax.lax.dot_general(ones, em, (((1,), (0,)), ((), ())),
                             preferred_element_type=jnp.float32)

    q = jax.lax.shift_right_logical(t, 3)
    r = jax.lax.bitwise_and(t, 7)
    iota_q = jax.lax.broadcasted_iota(jnp.int32, (_C // 8, _B), 0)
    oh_q = (iota_q == q[None, :]).astype(jnp.float32)
    g = jax.lax.dot_general(a2, oh_q, (((0,), (0,)), ((), ())),
                            preferred_element_type=jnp.float32)
    iota_r = jax.lax.broadcasted_iota(jnp.int32, (8, _B), 0)
    at = jnp.sum(jnp.where(iota_r == r[None, :], g, 0.0), axis=0)

    p = pe / s
    logp = jnp.log(p)
    omp = 1.0 - p
    loss = -at[None, :] * (omp * omp) * logp
    bsum = jnp.sum(loss, keepdims=True).reshape(1, 1)

    @pl.when(i == 0)
    def _init():
        out_ref[...] = jnp.zeros((1, 1), jnp.float32)

    acc = out_ref[...] + bsum
    out_ref[...] = jnp.where(i == nb - 1, acc * (1.0 / _N), acc)


@jax.jit
def kernel(inputs, targets, alpha):
    nb = _N // _B
    xt_view = inputs.T
    t3 = targets.reshape(nb, 1, _B)
    a2 = alpha.reshape(_C // 8, 8)
    out = pl.pallas_call(
        _focal_body,
        grid=(nb,),
        in_specs=[
            pl.BlockSpec((_C, _B), lambda i: (0, i)),
            pl.BlockSpec((1, 1, _B), lambda i: (i, 0, 0)),
            pl.BlockSpec((_C // 8, 8), lambda i: (0, 0)),
        ],
        out_specs=pl.BlockSpec((1, 1), lambda i: (0, 0)),
        out_shape=jax.ShapeDtypeStruct((1, 1), jnp.float32),
    )(xt_view, t3, a2)
    return out[0, 0]


# speedup vs baseline: 2.0691x; 1.1401x over previous
import functools

import jax
import jax.numpy as jnp
from jax.experimental import pallas as pl
from jax.experimental.pallas import tpu as pltpu

_N = 16384
_C = 1000
_GAMMA = 2.0
_B = 2048


def _focal_body(x_ref, t_ref, a_ref, out_ref):
    i = pl.program_id(0)
    nb = pl.num_programs(0)
    x = x_ref[...]
    t = t_ref[0, 0, :]
    a2 = a_ref[...]

    m = jnp.max(x, axis=0)
    e = jnp.exp(x - m[None, :])

    iota = jax.lax.broadcasted_iota(jnp.int32, x.shape, 0)
    onehot = iota == t[None, :]
    em = jnp.where(onehot, e, 0.0)

    ones = jnp.ones((1, _C), jnp.float32)
    s = jax.lax.dot_general(ones, e, (((1,), (0,)), ((), ())),
                            preferred_element_type=jnp.float32)
    pe = jax.lax.dot_general(ones, em, (((1,), (0,)), ((), ())),
                             preferred_element_type=jnp.float32)

    q = jax.lax.shift_right_logical(t, 3)
    r = jax.lax.bitwise_and(t, 7)
    iota_q = jax.lax.broadcasted_iota(jnp.int32, (_C // 8, _B), 0)
    oh_q = (iota_q == q[None, :]).astype(jnp.float32)
    g = jax.lax.dot_general(a2, oh_q, (((0,), (0,)), ((), ())),
                            preferred_element_type=jnp.float32)
    iota_r = jax.lax.broadcasted_iota(jnp.int32, (8, _B), 0)
    at = jnp.sum(jnp.where(iota_r == r[None, :], g, 0.0), axis=0)

    p = pe / s
    logp = jnp.log(p)
    omp = 1.0 - p
    loss = -at[None, :] * (omp * omp) * logp
    bsum = jnp.sum(loss, keepdims=True).reshape(1, 1)

    @pl.when(i == 0)
    def _init():
        out_ref[...] = jnp.zeros((1, 1), jnp.float32)

    acc = out_ref[...] + bsum
    out_ref[...] = jnp.where(i == nb - 1, acc * (1.0 / _N), acc)


@jax.jit
def kernel(inputs, targets, alpha):
    nb = _N // _B
    xt_view = inputs.T
    t3 = targets.reshape(nb, 1, _B)
    a2 = alpha.reshape(_C // 8, 8)
    out = pl.pallas_call(
        _focal_body,
        grid=(nb,),
        in_specs=[
            pl.BlockSpec((_C, _B), lambda i: (0, i)),
            pl.BlockSpec((1, 1, _B), lambda i: (i, 0, 0)),
            pl.BlockSpec((_C // 8, 8), lambda i: (0, 0)),
        ],
        out_specs=pl.BlockSpec((1, 1), lambda i: (0, 0)),
        out_shape=jax.ShapeDtypeStruct((1, 1), jnp.float32),
    )(xt_view, t3, a2)
    return out[0, 0]


# speedup vs baseline: 2.0923x; 1.0112x over previous
import functools

import jax
import jax.numpy as jnp
from jax.experimental import pallas as pl
from jax.experimental.pallas import tpu as pltpu

_N = 16384
_C = 1000
_GAMMA = 2.0
_B = 4096


def _focal_body(x_ref, t_ref, a_ref, out_ref):
    i = pl.program_id(0)
    nb = pl.num_programs(0)
    x = x_ref[...]
    t = t_ref[0, 0, :]
    a2 = a_ref[...]

    m = jnp.max(x, axis=0)
    e = jnp.exp(x - m[None, :])

    iota = jax.lax.broadcasted_iota(jnp.int32, x.shape, 0)
    onehot = iota == t[None, :]
    em = jnp.where(onehot, e, 0.0)

    ones = jnp.ones((1, _C), jnp.float32)
    s = jax.lax.dot_general(ones, e, (((1,), (0,)), ((), ())),
                            preferred_element_type=jnp.float32)
    pe = jax.lax.dot_general(ones, em, (((1,), (0,)), ((), ())),
                             preferred_element_type=jnp.float32)

    q = jax.lax.shift_right_logical(t, 3)
    r = jax.lax.bitwise_and(t, 7)
    iota_q = jax.lax.broadcasted_iota(jnp.int32, (_C // 8, _B), 0)
    oh_q = (iota_q == q[None, :]).astype(jnp.float32)
    g = jax.lax.dot_general(a2, oh_q, (((0,), (0,)), ((), ())),
                            preferred_element_type=jnp.float32)
    iota_r = jax.lax.broadcasted_iota(jnp.int32, (8, _B), 0)
    at = jnp.sum(jnp.where(iota_r == r[None, :], g, 0.0), axis=0)

    p = pe / s
    logp = jnp.log(p)
    omp = 1.0 - p
    loss = -at[None, :] * (omp * omp) * logp
    bsum = jnp.sum(loss, keepdims=True).reshape(1, 1)

    @pl.when(i == 0)
    def _init():
        out_ref[...] = jnp.zeros((1, 1), jnp.float32)

    acc = out_ref[...] + bsum
    out_ref[...] = jnp.where(i == nb - 1, acc * (1.0 / _N), acc)


@jax.jit
def kernel(inputs, targets, alpha):
    nb = _N // _B
    xt_view = inputs.T
    t3 = targets.reshape(nb, 1, _B)
    a2 = alpha.reshape(_C // 8, 8)
    out = pl.pallas_call(
        _focal_body,
        grid=(nb,),
        in_specs=[
            pl.BlockSpec((_C, _B), lambda i: (0, i)),
            pl.BlockSpec((1, 1, _B), lambda i: (i, 0, 0)),
            pl.BlockSpec((_C // 8, 8), lambda i: (0, 0)),
        ],
        out_specs=pl.BlockSpec((1, 1), lambda i: (0, 0)),
        out_shape=jax.ShapeDtypeStruct((1, 1), jnp.float32),
    )(xt_view, t3, a2)
    return out[0, 0]
